# baseline (speedup 1.0000x reference)
import functools
import math

import jax
import jax.numpy as jnp
from jax import lax
from jax.experimental import pallas as pl
from jax.experimental.pallas import tpu as pltpu
from jax.experimental.pallas import tpu_sc as plsc

_SPARSE_RATIO = 0.6
_L = 16


def _lane(x, j):
    idx = jnp.full((_L,), j, jnp.int32)
    return x.at[idx].get(mode="promise_in_bounds")


def _lane_v(x, idx_splat):
    idx = jnp.minimum(idx_splat, _L - 1)
    return x.at[idx].get(mode="promise_in_bounds")


def _sc_mask_body(sa_hbm, m2_hbm, m3_hbm, mask_hbm,
                  sa_v, m2_v, m3_v, bits_v, hist_v, mask_v,
                  *, n, num_keep, num_batches):
    nc = 2
    wid = lax.axis_index("s") * nc + lax.axis_index("c")
    nchunks = n // _L
    iota = lax.iota(jnp.int32, _L)
    big = jnp.full((_L,), 2 ** 30, jnp.int32)
    ones_i = jnp.ones((_L,), jnp.int32)

    @pl.when(wid < num_batches)
    def _():
        b = wid
        pltpu.sync_copy(sa_hbm.at[b], sa_v)
        pltpu.sync_copy(m2_hbm.at[b], m2_v)
        pltpu.sync_copy(m3_hbm.at[b], m3_v)

        def mm_body(i, carry):
            mn1, mx1, mn2, mx2, mn3, mx3 = carry
            sl = pl.ds(i * _L, _L)
            v1 = plsc.bitcast(sa_v[sl], jnp.int32)
            v2 = plsc.bitcast(m2_v[sl], jnp.int32)
            v3 = plsc.bitcast(m3_v[sl], jnp.int32)
            return (jnp.minimum(mn1, v1), jnp.maximum(mx1, v1),
                    jnp.minimum(mn2, v2), jnp.maximum(mx2, v2),
                    jnp.minimum(mn3, v3), jnp.maximum(mx3, v3))

        neg = jnp.full((_L,), -1, jnp.int32)
        r = lax.fori_loop(0, nchunks, mm_body, (big, neg, big, neg, big, neg))

        def _redmax(x):
            return _lane(plsc.cummax(x), _L - 1)

        def _redmin(x):
            return -_redmax(-x)

        mn1 = plsc.bitcast(_redmin(r[0]), jnp.float32)
        mx1 = plsc.bitcast(_redmax(r[1]), jnp.float32)
        mn2 = plsc.bitcast(_redmin(r[2]), jnp.float32)
        mx2 = plsc.bitcast(_redmax(r[3]), jnp.float32)
        mn3 = plsc.bitcast(_redmin(r[4]), jnp.float32)
        mx3 = plsc.bitcast(_redmax(r[5]), jnp.float32)
        d1 = mx1 - mn1 + 1e-08
        d2 = mx2 - mn2 + 1e-08
        d3 = mx3 - mn3 + 1e-08

        def sc_body(i, carry):
            sl = pl.ds(i * _L, _L)
            s1 = (sa_v[sl] - mn1) / d1
            s2 = (m2_v[sl] - mn2) / d2
            s3 = (m3_v[sl] - mn3) / d3
            score = (s1 + s2 + s3) / 3.0
            bits_v[sl] = plsc.bitcast(score, jnp.int32)
            return carry

        lax.fori_loop(0, nchunks, sc_body, 0)

        rk = jnp.full((_L,), num_keep, jnp.int32)
        act_total = jnp.full((_L,), n, jnp.int32)
        pref = jnp.zeros((_L,), jnp.int32)
        for shift, dbits in ((22, 8), (14, 8), (6, 8), (0, 6)):
            nbins = 1 << dbits
            hishift = shift + dbits

            def z_body(i, carry):
                hist_v[pl.ds(i * _L, _L)] = jnp.zeros((_L,), jnp.int32)
                return carry

            lax.fori_loop(0, 256, z_body, 0)

            def h_body(i, carry, shift=shift, hishift=hishift, pref=pref,
                       nbins=nbins):
                sl = pl.ds(i * _L, _L)
                v = bits_v[sl]
                act = lax.shift_right_logical(v, hishift) == pref
                d = lax.shift_right_logical(v, shift) & (nbins - 1)
                plsc.addupdate_scatter(hist_v, [iota * 256 + d], ones_i,
                                       mask=act)
                return carry

            lax.fori_loop(0, nchunks, h_body, 0)

            thresh = act_total - rk + 1
            csum = jnp.zeros((_L,), jnp.int32)
            dstar = big
            c_at = big
            c_before = big
            for j in range(nbins // _L):
                tot = hist_v[pl.ds(j * _L, _L)]
                for l in range(1, _L):
                    tot = tot + hist_v[pl.ds(l * 256 + j * _L, _L)]
                totcum = plsc.cumsum(tot) + csum
                hit = totcum >= thresh
                lane = plsc.all_reduce_ffs(hit)
                found = (lane < _L) & (dstar >= big)
                dstar = jnp.where(found, j * _L + lane, dstar)
                c_at = jnp.where(found, _lane_v(totcum, lane), c_at)
                c_before = jnp.where(found,
                                     c_at - _lane_v(tot, lane), c_before)
                csum = _lane(totcum, _L - 1)

            rk = rk - (act_total - c_at)
            act_total = c_at - c_before
            pref = (pref << dbits) | dstar

        tbits = pref
        need = rk

        def t_body(i, carry):
            cum, cidx = carry
            v = bits_v[pl.ds(i * _L, _L)]
            eqm = v == tbits
            cnts = plsc.cumsum(jnp.where(eqm, 1, 0).astype(jnp.int32)) + cum
            hit = eqm & (cnts == need)
            lane = plsc.all_reduce_ffs(hit)
            found = (lane < _L) & (cidx >= big)
            cidx = jnp.where(found, i * _L + lane, cidx)
            cum = _lane(cnts, _L - 1)
            return cum, cidx

        _, cidx = lax.fori_loop(0, nchunks, t_body,
                                (jnp.zeros((_L,), jnp.int32), big))

        def m_body(i, carry):
            sl = pl.ds(i * _L, _L)
            v = bits_v[sl]
            keep = (v > tbits) | ((v == tbits) & (i * _L + iota <= cidx))
            mask_v[sl] = jnp.where(keep, 1.0, 0.0).astype(jnp.float32)
            return carry

        lax.fori_loop(0, nchunks, m_body, 0)
        pltpu.sync_copy(mask_v, mask_hbm.at[b])


def _sc_mask(sa, m2, m3):
    B, N = sa.shape
    num_keep = max(1, math.ceil(N * _SPARSE_RATIO))
    mesh = plsc.VectorSubcoreMesh(core_axis_name="c", subcore_axis_name="s")
    body = functools.partial(_sc_mask_body, n=N, num_keep=num_keep,
                             num_batches=B)
    return pl.kernel(
        body,
        out_type=jax.ShapeDtypeStruct((B, N), jnp.float32),
        mesh=mesh,
        compiler_params=pltpu.CompilerParams(needs_layout_passes=False),
        scratch_types=[
            pltpu.VMEM((N,), jnp.float32),
            pltpu.VMEM((N,), jnp.float32),
            pltpu.VMEM((N,), jnp.float32),
            pltpu.VMEM((N,), jnp.int32),
            pltpu.VMEM((256 * _L,), jnp.int32),
            pltpu.VMEM((N,), jnp.float32),
        ],
    )(sa, m2, m3)


def _mul_body(mask_ref, tok_ref, out_ref, maskT_ref, *, blk_n):
    b = pl.program_id(0)
    j = pl.program_id(1)

    @pl.when((b == 0) & (j == 0))
    def _():
        maskT_ref[...] = mask_ref[...].T

    cols = maskT_ref[pl.ds(j * blk_n, blk_n), :]
    m = cols[:, 3:4]
    for bi in (2, 1, 0):
        m = jnp.where(b == bi, cols[:, bi:bi + 1], m)
    out_ref[...] = tok_ref[...] * m[None, :, :]


def kernel(tokens, self_attention, cross_attention_m2, cross_attention_m3):
    B, N, C = tokens.shape
    mask = _sc_mask(self_attention, cross_attention_m2, cross_attention_m3)
    blk_n = 2048
    body = functools.partial(_mul_body, blk_n=blk_n)
    masked = pl.pallas_call(
        body,
        grid=(B, N // blk_n),
        in_specs=[
            pl.BlockSpec((B, N), lambda b, j: (0, 0)),
            pl.BlockSpec((1, blk_n, C), lambda b, j: (b, j, 0)),
        ],
        out_specs=pl.BlockSpec((1, blk_n, C), lambda b, j: (b, j, 0)),
        out_shape=jax.ShapeDtypeStruct((B, N, C), tokens.dtype),
        scratch_shapes=[pltpu.VMEM((N, B), jnp.float32)],
    )(mask, tokens)
    return masked, mask


# speedup vs baseline: 1.8677x; 1.8677x over previous
import functools
import math

import jax
import jax.numpy as jnp
from jax import lax
from jax.experimental import pallas as pl
from jax.experimental.pallas import tpu as pltpu

_SPARSE_RATIO = 0.6


def _fused_body(sa_ref, m2_ref, m3_ref, tok_ref, out_ref, mask_ref, maskT_ref,
                *, num_keep, blk_n, n):
    b = pl.program_id(0)
    j = pl.program_id(1)

    @pl.when((b == 0) & (j == 0))
    def _compute_mask():
        def norm(s):
            mn = jnp.min(s, axis=-1, keepdims=True)
            mx = jnp.max(s, axis=-1, keepdims=True)
            return (s - mn) / (mx - mn + 1e-08)

        score = (norm(sa_ref[...]) + norm(m2_ref[...]) + norm(m3_ref[...])) / 3.0
        bits = lax.bitcast_convert_type(score, jnp.int32)
        nb = score.shape[0]
        lo0 = jnp.zeros((nb, 1), jnp.int32)
        hi0 = jnp.full((nb, 1), 0x3F800000, jnp.int32)

        def bis(_, carry):
            lo, hi = carry
            mid = lo + (hi - lo) // 2
            cnt = jnp.sum((bits >= mid).astype(jnp.int32), axis=-1, keepdims=True)
            ge = cnt >= num_keep
            return jnp.where(ge, mid, lo), jnp.where(ge, hi, mid)

        tbits, _ = lax.fori_loop(0, 30, bis, (lo0, hi0))
        gt = bits > tbits
        eq = bits == tbits
        need = num_keep - jnp.sum(gt.astype(jnp.int32), axis=-1, keepdims=True)
        idx = lax.broadcasted_iota(jnp.int32, score.shape, 1)
        eqcum = eq.astype(jnp.int32)
        d = 1
        while d < n:
            rolled = pltpu.roll(eqcum, d, axis=1)
            eqcum = eqcum + jnp.where(idx >= d, rolled, 0)
            d *= 2
        mask = (gt | (eq & (eqcum <= need))).astype(jnp.float32)
        mask_ref[...] = mask
        maskT_ref[...] = mask.T

    off = j * blk_n
    cols = maskT_ref[pl.ds(off, blk_n), :]
    m = cols[:, 3:4]
    for bi in (2, 1, 0):
        m = jnp.where(b == bi, cols[:, bi:bi + 1], m)
    out_ref[...] = tok_ref[...] * m[None, :, :]


def kernel(tokens, self_attention, cross_attention_m2, cross_attention_m3):
    B, N, C = tokens.shape
    num_keep = max(1, math.ceil(N * _SPARSE_RATIO))
    blk_n = 2048
    nbpb = N // blk_n
    body = functools.partial(_fused_body, num_keep=num_keep, blk_n=blk_n, n=N)
    masked, mask = pl.pallas_call(
        body,
        grid=(B, nbpb),
        in_specs=[
            pl.BlockSpec((B, N), lambda b, j: (0, 0)),
            pl.BlockSpec((B, N), lambda b, j: (0, 0)),
            pl.BlockSpec((B, N), lambda b, j: (0, 0)),
            pl.BlockSpec((1, blk_n, C), lambda b, j: (b, j, 0)),
        ],
        out_specs=[
            pl.BlockSpec((1, blk_n, C), lambda b, j: (b, j, 0)),
            pl.BlockSpec((B, N), lambda b, j: (0, 0)),
        ],
        out_shape=[
            jax.ShapeDtypeStruct((B, N, C), tokens.dtype),
            jax.ShapeDtypeStruct((B, N), jnp.float32),
        ],
        scratch_shapes=[pltpu.VMEM((N, B), jnp.float32)],
    )(self_attention, cross_attention_m2, cross_attention_m3, tokens)
    return masked, mask


# speedup vs baseline: 2.0271x; 1.0853x over previous
import functools
import math

import jax
import jax.numpy as jnp
from jax import lax
from jax.experimental import pallas as pl
from jax.experimental.pallas import tpu as pltpu

_SPARSE_RATIO = 0.6


def _body(sa_ref, m2_ref, m3_ref, tok_hbm, out_hbm, mask_ref,
          ibuf, obuf, maskT_ref, isem, osem,
          *, num_keep, blk_n, n, nbuf, nblocks, nbpb):
    def norm(s):
        mn = jnp.min(s, axis=-1, keepdims=True)
        mx = jnp.max(s, axis=-1, keepdims=True)
        return (s - mn) / (mx - mn + 1e-08)

    def fetch(s):
        b, jo = s // nbpb, s % nbpb
        pltpu.make_async_copy(
            tok_hbm.at[b, pl.ds(jo * blk_n, blk_n), :],
            ibuf.at[s % nbuf], isem.at[s % nbuf]).start()

    for s in range(nbuf):
        fetch(s)

    score = (norm(sa_ref[...]) + norm(m2_ref[...]) + norm(m3_ref[...])) / 3.0
    bits = lax.bitcast_convert_type(score, jnp.int32)
    nb = score.shape[0]
    lo0 = jnp.zeros((nb, 1), jnp.int32)
    hi0 = jnp.full((nb, 1), 0x3F800000, jnp.int32)

    def bis(_, carry):
        lo, hi = carry
        mid = lo + (hi - lo) // 2
        cnt = jnp.sum((bits >= mid).astype(jnp.int32), axis=-1, keepdims=True)
        ge = cnt >= num_keep
        return jnp.where(ge, mid, lo), jnp.where(ge, hi, mid)

    tbits, _ = lax.fori_loop(0, 30, bis, (lo0, hi0))
    gt = bits > tbits
    eq = bits == tbits
    need = num_keep - jnp.sum(gt.astype(jnp.int32), axis=-1, keepdims=True)
    idx = lax.broadcasted_iota(jnp.int32, score.shape, 1)
    eqcum = eq.astype(jnp.int32)
    d = 1
    while d < n:
        rolled = pltpu.roll(eqcum, d, axis=1)
        eqcum = eqcum + jnp.where(idx >= d, rolled, 0)
        d *= 2
    mask = (gt | (eq & (eqcum <= need))).astype(jnp.float32)
    mask_ref[...] = mask
    maskT_ref[...] = mask.T

    for s in range(nblocks):
        slot = s % nbuf
        b, jo = s // nbpb, s % nbpb
        if s >= nbuf:
            bp, jp = (s - nbuf) // nbpb, (s - nbuf) % nbpb
            pltpu.make_async_copy(
                obuf.at[slot],
                out_hbm.at[bp, pl.ds(jp * blk_n, blk_n), :],
                osem.at[slot]).wait()
        pltpu.make_async_copy(
            tok_hbm.at[b, pl.ds(jo * blk_n, blk_n), :],
            ibuf.at[slot], isem.at[slot]).wait()
        m = maskT_ref[pl.ds(jo * blk_n, blk_n), b:b + 1]
        obuf[slot] = ibuf[slot] * m
        pltpu.make_async_copy(
            obuf.at[slot],
            out_hbm.at[b, pl.ds(jo * blk_n, blk_n), :],
            osem.at[slot]).start()
        if s + nbuf < nblocks:
            fetch(s + nbuf)

    for s in range(max(0, nblocks - nbuf), nblocks):
        slot = s % nbuf
        b, jo = s // nbpb, s % nbpb
        pltpu.make_async_copy(
            obuf.at[slot],
            out_hbm.at[b, pl.ds(jo * blk_n, blk_n), :],
            osem.at[slot]).wait()


def kernel(tokens, self_attention, cross_attention_m2, cross_attention_m3):
    B, N, C = tokens.shape
    num_keep = max(1, math.ceil(N * _SPARSE_RATIO))
    blk_n = 512
    nbuf = 6
    nbpb = N // blk_n
    nblocks = B * nbpb
    body = functools.partial(_body, num_keep=num_keep, blk_n=blk_n, n=N,
                             nbuf=nbuf, nblocks=nblocks, nbpb=nbpb)
    masked, mask = pl.pallas_call(
        body,
        in_specs=[
            pl.BlockSpec(memory_space=pltpu.VMEM),
            pl.BlockSpec(memory_space=pltpu.VMEM),
            pl.BlockSpec(memory_space=pltpu.VMEM),
            pl.BlockSpec(memory_space=pl.ANY),
        ],
        out_specs=[
            pl.BlockSpec(memory_space=pl.ANY),
            pl.BlockSpec(memory_space=pltpu.VMEM),
        ],
        out_shape=[
            jax.ShapeDtypeStruct((B, N, C), tokens.dtype),
            jax.ShapeDtypeStruct((B, N), jnp.float32),
        ],
        scratch_shapes=[
            pltpu.VMEM((nbuf, blk_n, C), jnp.float32),
            pltpu.VMEM((nbuf, blk_n, C), jnp.float32),
            pltpu.VMEM((N, B), jnp.float32),
            pltpu.SemaphoreType.DMA((nbuf,)),
            pltpu.SemaphoreType.DMA((nbuf,)),
        ],
    )(self_attention, cross_attention_m2, cross_attention_m3, tokens)
    return masked, mask
